# baseline (speedup 1.0000x reference)
import functools

import jax
import jax.numpy as jnp
from jax import lax
from jax.experimental import pallas as pl
from jax.experimental.pallas import tpu as pltpu
from jax.experimental.pallas import tpu_sc as plsc

_TOP_K = 64


def _scores(x):
    b, c, t, pn = x.shape
    xr = jnp.transpose(x, (0, 3, 2, 1)).reshape(b, pn, -1)
    d_k = xr.shape[-1]
    sim = jnp.matmul(xr, jnp.swapaxes(xr, -2, -1)) * (d_k ** -0.5)
    attn = jax.nn.softmax(sim, axis=-1)
    return jnp.mean(attn, axis=-1)


def _topk_body(scores_ref, idx_ref):
    s = scores_ref[...]
    bsz, pn = s.shape
    col = lax.broadcasted_iota(jnp.int32, (bsz, pn), 1)
    kcol = lax.broadcasted_iota(jnp.int32, (bsz, _TOP_K), 1)
    rowoff = lax.broadcasted_iota(jnp.int32, (bsz, _TOP_K), 0) * pn

    def step(k, carry):
        s, acc = carry
        m = jnp.max(s, axis=1, keepdims=True)
        cand = jnp.where(s == m, col, pn)
        i = jnp.min(cand, axis=1, keepdims=True)
        acc = jnp.where(kcol == k, i, acc)
        s = jnp.where(col == i, -jnp.inf, s)
        return s, acc

    _, acc = lax.fori_loop(
        0, _TOP_K, step, (s, jnp.zeros((bsz, _TOP_K), jnp.int32))
    )
    del rowoff
    idx_ref[...] = acc


def _topk(scores):
    bsz, pn = scores.shape
    return pl.pallas_call(
        _topk_body,
        out_shape=jax.ShapeDtypeStruct((bsz, _TOP_K), jnp.int32),
    )(scores)


def _make_gather(b, pn, t, h, w):
    info = plsc.get_sparse_core_info()
    nw = info.num_cores * info.num_subcores
    n_idx = b * _TOP_K
    per_w = n_idx // nw
    assert _TOP_K % per_w == 0
    mesh = plsc.VectorSubcoreMesh(core_axis_name="c", subcore_axis_name="s")

    @functools.partial(
        pl.kernel,
        mesh=mesh,
        out_type=jax.ShapeDtypeStruct((n_idx, t * h * w), jnp.float32),
        scratch_types=[
            pltpu.VMEM((per_w,), jnp.int32),
            pltpu.VMEM((per_w, t * h * w), jnp.float32),
            pltpu.SemaphoreType.DMA,
        ],
    )
    def gather(table_hbm, idx_hbm, out_hbm, idx_v, rows_v, sem):
        wid = lax.axis_index("s") * info.num_cores + lax.axis_index("c")
        base = wid * per_w
        bidx = base // _TOP_K
        pltpu.sync_copy(idx_hbm.at[pl.ds(base, per_w)], idx_v)
        pltpu.async_copy(table_hbm.at[bidx].at[idx_v], rows_v, sem).wait()
        pltpu.sync_copy(rows_v, out_hbm.at[pl.ds(base, per_w)])

    return gather


def kernel(x, traj_map):
    b, pn, t, h, w = traj_map.shape
    scores = _scores(x)
    lidx = _topk(scores).reshape(b * _TOP_K)
    table = traj_map.reshape(b, pn, t * h * w)
    rows = _make_gather(b, pn, t, h, w)(table, lidx)
    return rows.reshape(b, _TOP_K, t, h, w)
